# Initial kernel scaffold; baseline (speedup 1.0000x reference)
import jax
import jax.numpy as jnp
from jax.experimental import pallas as pl


def kernel(x, edge_index, W1, b1, g1, be1, W2, b2, g2, be2, W3, b3, g3, be3, fcW, fcb):
    raise NotImplementedError("write your pallas kernel here")


# speedup vs baseline: 8.7037x; 8.7037x over previous
import functools

import jax
import jax.numpy as jnp
from jax import lax
from jax.experimental import pallas as pl
from jax.experimental.pallas import tpu as pltpu
from jax.experimental.pallas import tpu_sc as plsc

N = 10000
D = 128
H = 128
E = 320000

NC = 2
NS = 16
NT = NC * NS

NP = 10240
NB = NP // 128
EPT = 10112
KB = EPT // 128
EP = NT * EPT
RPT = NP // NS

_mesh = plsc.VectorSubcoreMesh(core_axis_name="c", subcore_axis_name="s")
_sc_params = pltpu.CompilerParams(needs_layout_passes=False)


@functools.partial(
    pl.kernel,
    out_type=jax.ShapeDtypeStruct((NT, NP), jnp.float32),
    mesh=_mesh,
    compiler_params=_sc_params,
    scratch_types=[
        pltpu.VMEM((EPT,), jnp.int32),
        pltpu.VMEM((NP,), jnp.float32),
    ],
)
def _deg_call(dst_hbm, degp_hbm, dst_v, deg_v):
    c = lax.axis_index("c")
    s = lax.axis_index("s")
    wid = c * NS + s

    def _zero(i, _):
        deg_v[pl.ds(i * 16, 16)] = jnp.zeros((16,), jnp.float32)
        return _

    lax.fori_loop(0, NP // 16, _zero, None)
    pltpu.sync_copy(dst_hbm.at[wid], dst_v)
    ones = jnp.ones((16,), jnp.float32)

    def _count(i, _):
        idx = dst_v[pl.ds(i * 16, 16)]
        plsc.addupdate_scatter(deg_v, [idx], ones)
        return _

    lax.fori_loop(0, EPT // 16, _count, None)
    pltpu.sync_copy(deg_v, degp_hbm.at[wid])


@functools.partial(
    pl.kernel,
    out_type=jax.ShapeDtypeStruct((NC, NP, 128), jnp.float32),
    mesh=_mesh,
    compiler_params=_sc_params,
    scratch_types=[
        pltpu.VMEM((KB, 128), jnp.int32),
        pltpu.VMEM((KB, 128), jnp.int32),
        pltpu.VMEM((128, 128), jnp.float32),
        pltpu.VMEM_SHARED((NP, 128), jnp.float32),
        pltpu.SemaphoreType.DMA,
    ],
)
def _edge_call(hs_hbm, src_hbm, dst_hbm, zer_hbm, out_hbm,
               src_v, dst_v, rows_v, acc_sh, sem):
    c = lax.axis_index("c")
    s = lax.axis_index("s")
    wid = c * NS + s

    pltpu.sync_copy(zer_hbm, acc_sh.at[pl.ds(s * RPT, RPT)])
    pltpu.sync_copy(src_hbm.at[wid], src_v)
    pltpu.sync_copy(dst_hbm.at[wid], dst_v)
    plsc.subcore_barrier()

    def _block(j, _):
        pltpu.async_copy(hs_hbm.at[src_v.at[j]], rows_v, sem).wait()
        pltpu.sync_copy(rows_v, acc_sh.at[dst_v.at[j]], add=True)
        return _

    lax.fori_loop(0, KB, _block, None)
    plsc.subcore_barrier()
    pltpu.sync_copy(acc_sh.at[pl.ds(s * RPT, RPT)],
                    out_hbm.at[c, pl.ds(s * RPT, RPT)])


def _dinv_body(degp_ref, dinv_ref):
    deg = jnp.sum(degp_ref[...], axis=0) + 1.0
    r = lax.rsqrt(deg)
    row = lax.broadcasted_iota(jnp.int32, (NB, 128), 0)
    col = lax.broadcasted_iota(jnp.int32, (NB, 128), 1)
    dinv_ref[...] = jnp.where(row * 128 + col < N, r, 0.0)


def _mm_body(x_ref, w_ref, dv_ref, o_ref):
    h = jnp.dot(x_ref[...], w_ref[...], preferred_element_type=jnp.float32)
    o_ref[...] = h * dv_ref[...]


def _pre_body(p_ref, hs_ref, dv_ref, pre_ref, sums_ref):
    t = (p_ref[0] + p_ref[1] + hs_ref[...]) * dv_ref[...]
    pre_ref[...] = t

    @pl.when(pl.program_id(0) == 0)
    def _():
        sums_ref[...] = jnp.zeros_like(sums_ref)

    sums_ref[0:1, :] += jnp.sum(t, axis=0, keepdims=True)
    sums_ref[1:2, :] += jnp.sum(t * t, axis=0, keepdims=True)


def _bn_stats(sums_ref, g_ref, be_ref):
    mu = sums_ref[0:1, :] / N
    var = sums_ref[1:2, :] / N - mu * mu
    scale = g_ref[...] * lax.rsqrt(var + 1e-5)
    shift = be_ref[...] - mu * scale
    return scale, shift


def _bn_mm_body(pre_ref, sums_ref, g_ref, be_ref, w_ref, dv_ref, o_ref):
    scale, shift = _bn_stats(sums_ref, g_ref, be_ref)
    y = jnp.maximum(pre_ref[...] * scale + shift, 0.0)
    h = jnp.dot(y, w_ref[...], preferred_element_type=jnp.float32)
    o_ref[...] = h * dv_ref[...]


def _head_body(pre_ref, sums_ref, g_ref, be_ref, w_ref, fb_ref, o_ref):
    scale, shift = _bn_stats(sums_ref, g_ref, be_ref)
    y = jnp.maximum(pre_ref[...] * scale + shift, 0.0)
    t = jnp.dot(y, w_ref[...], preferred_element_type=jnp.float32) + fb_ref[...]
    o_ref[...] = jax.nn.sigmoid(t)


_f32 = jnp.float32
_blk = pl.BlockSpec((128, 128), lambda i: (i, 0))
_col = pl.BlockSpec((128, 1), lambda i: (i, 0))
_full = pl.BlockSpec((128, 128), lambda i: (0, 0))
_vec = pl.BlockSpec((1, 128), lambda i: (0, 0))
_sums = pl.BlockSpec((8, 128), lambda i: (0, 0))


def _dinv_call(degp):
    return pl.pallas_call(
        _dinv_body,
        out_shape=jax.ShapeDtypeStruct((NB, 128), _f32),
    )(degp.reshape(NT, NB, 128))


def _mm_call(x, w, dv):
    return pl.pallas_call(
        _mm_body, grid=(NB,),
        in_specs=[_blk, _full, _col],
        out_specs=_blk,
        out_shape=jax.ShapeDtypeStruct((NP, 128), _f32),
    )(x, w, dv)


def _pre_call(p, hs, dv):
    return pl.pallas_call(
        _pre_body, grid=(NB,),
        in_specs=[pl.BlockSpec((NC, 128, 128), lambda i: (0, i, 0)), _blk, _col],
        out_specs=[_blk, _sums],
        out_shape=[jax.ShapeDtypeStruct((NP, 128), _f32),
                   jax.ShapeDtypeStruct((8, 128), _f32)],
    )(p, hs, dv)


def _bn_mm_call(pre, sums, g, be, w, dv):
    return pl.pallas_call(
        _bn_mm_body, grid=(NB,),
        in_specs=[_blk, _sums, _vec, _vec, _full, _col],
        out_specs=_blk,
        out_shape=jax.ShapeDtypeStruct((NP, 128), _f32),
    )(pre, sums, g.reshape(1, 128), be.reshape(1, 128), w, dv)


def _head_call(pre, sums, g, be, wp, fb):
    return pl.pallas_call(
        _head_body, grid=(NB,),
        in_specs=[_blk, _sums, _vec, _vec, _full, _vec],
        out_specs=_blk,
        out_shape=jax.ShapeDtypeStruct((NP, 128), _f32),
    )(pre, sums, g.reshape(1, 128), be.reshape(1, 128), wp, fb)


def kernel(x, edge_index, W1, b1, g1, be1, W2, b2, g2, be2, W3, b3, g3, be3,
           fcW, fcb):
    del b1, b2, b3
    src = edge_index[0]
    dst = edge_index[1]
    fill = jnp.full((EP - E,), N, jnp.int32)
    src_p = jnp.concatenate([src, fill])
    dst_p = jnp.concatenate([dst, fill])
    src3 = src_p.reshape(NT, KB, 128)
    dst3 = dst_p.reshape(NT, KB, 128)
    dst2 = dst_p.reshape(NT, EPT)
    x_pad = jnp.pad(x, ((0, NP - N), (0, 0)))
    zer = jnp.zeros((RPT, 128), _f32)
    fcWp = jnp.pad(fcW, ((0, 0), (0, 127)))
    fbv = jnp.broadcast_to(fcb, (128,)).reshape(1, 128)

    degp = _deg_call(dst2)
    dinv = _dinv_call(degp).reshape(NP, 1)

    hs = _mm_call(x_pad, W1, dinv)
    p1 = _edge_call(hs, src3, dst3, zer)
    pre1, sums1 = _pre_call(p1, hs, dinv)

    hs2 = _bn_mm_call(pre1, sums1, g1, be1, W2, dinv)
    p2 = _edge_call(hs2, src3, dst3, zer)
    pre2, sums2 = _pre_call(p2, hs2, dinv)

    hs3 = _bn_mm_call(pre2, sums2, g2, be2, W3, dinv)
    p3 = _edge_call(hs3, src3, dst3, zer)
    pre3, sums3 = _pre_call(p3, hs3, dinv)

    res = _head_call(pre3, sums3, g3, be3, fcWp, fbv)
    return res[:N, :1]
